# baseline (speedup 1.0000x reference)
import functools

import jax
import jax.numpy as jnp
from jax import lax
from jax.experimental import pallas as pl
from jax.experimental.pallas import tpu as pltpu
from jax.experimental.pallas import tpu_sc as plsc

_SCORE_T = 0.001
_IOU_T = 0.7
_MAX_DET = 300
_PRE_NMS = 4096
_BLK = 256

_NTILES = 32
_CHUNK = 50176
_NFLATP = _NTILES * _CHUNK
_NBUCKETS = 65536
_CAP = 16384
_ROWS = _CHUNK // 128
_SLAB_ROWS = _ROWS // 2
_SLAB = _SLAB_ROWS * 128


def _sc_hist_body(keys_hbm, hist_hbm, chunk_v, hist_v, sem):
    cid = lax.axis_index("c")
    sid = lax.axis_index("s")
    wid = sid * 2 + cid
    base = wid * _CHUNK

    zeros16 = jnp.zeros((16,), jnp.int32)

    def zbody(j, carry):
        hist_v[pl.ds(j * 16, 16)] = zeros16
        return carry

    lax.fori_loop(0, _NBUCKETS // 16, zbody, 0)

    pltpu.sync_copy(keys_hbm.at[pl.ds(base, _CHUNK)], chunk_v)

    ones16 = jnp.ones((16,), jnp.int32)

    def body(j, carry):
        k = chunk_v[pl.ds(j * 16, 16)]
        bkt = (k >> 16) + 32768
        plsc.addupdate_scatter(hist_v, [bkt], ones16)
        return carry

    lax.fori_loop(0, _CHUNK // 16, body, 0)
    pltpu.sync_copy(hist_v, hist_hbm.at[wid])


def _sc_hist(keys_p):
    mesh = plsc.VectorSubcoreMesh(core_axis_name="c", subcore_axis_name="s")
    return pl.kernel(
        _sc_hist_body,
        out_type=jax.ShapeDtypeStruct((_NTILES, _NBUCKETS), jnp.int32),
        mesh=mesh,
        scratch_types=[
            pltpu.VMEM((_CHUNK,), jnp.int32),
            pltpu.VMEM((_NBUCKETS,), jnp.int32),
            pltpu.SemaphoreType.DMA,
        ],
        compiler_params=pltpu.CompilerParams(needs_layout_passes=False),
    )(keys_p)


def _sc_compact_body(keys_hbm, kedge_hbm, bases_hbm, out_hbm,
                     slab_v, dest_v, src_v, tvec_v, bvec_v, sem):
    cid = lax.axis_index("c")
    sid = lax.axis_index("s")
    wid = sid * 2 + cid
    base = wid * _CHUNK

    pltpu.sync_copy(kedge_hbm, tvec_v)
    pltpu.sync_copy(bases_hbm.at[wid], bvec_v)
    tvec = tvec_v[...]
    bvec = bvec_v[...]
    lane = lax.iota(jnp.int32, 16)

    cnt = jnp.int32(0)
    for half in range(2):
        so = half * _SLAB
        pltpu.sync_copy(keys_hbm.at[pl.ds(base + so, _SLAB)], slab_v)

        def rowbody(r, cnt, so=so):
            for q in range(8):
                v = slab_v[pl.ds(r * 128 + q * 16, 16)]
                msk = v >= tvec
                inc = jnp.where(msk, 1, 0).astype(jnp.int32)
                pfx = plsc.cumsum(inc)
                npop = jnp.sum(inc)
                rank = bvec + cnt + pfx - 1
                dest = jnp.where(msk, rank, _CAP + lane)
                dest_v[r, pl.ds(q * 16, 16)] = dest
                src_v[r, pl.ds(q * 16, 16)] = base + so + r * 128 + q * 16 + lane
                cnt = cnt + npop
            return cnt

        cnt = lax.fori_loop(0, _SLAB_ROWS, rowbody, cnt)

        def fire(r, carry):
            pltpu.async_copy(src_v.at[r], out_hbm.at[dest_v.at[r]], sem)
            return carry

        lax.fori_loop(0, _SLAB_ROWS, fire, 0)
        pltpu.make_async_copy(keys_hbm.at[pl.ds(0, _SLAB)], slab_v, sem).wait()


def _sc_compact(keys_p, kedge16, bases_rep):
    mesh = plsc.VectorSubcoreMesh(core_axis_name="c", subcore_axis_name="s")
    return pl.kernel(
        _sc_compact_body,
        out_type=jax.ShapeDtypeStruct((_CAP + 128,), jnp.int32),
        mesh=mesh,
        scratch_types=[
            pltpu.VMEM((_SLAB,), jnp.int32),
            pltpu.VMEM((_SLAB_ROWS, 128), jnp.int32),
            pltpu.VMEM((_SLAB_ROWS, 128), jnp.int32),
            pltpu.VMEM((16,), jnp.int32),
            pltpu.VMEM((16,), jnp.int32),
            pltpu.SemaphoreType.DMA,
        ],
        compiler_params=pltpu.CompilerParams(needs_layout_passes=False),
    )(keys_p, kedge16, bases_rep)


def _select_topk(flat):
    n = flat.shape[0]
    flat_p = jnp.concatenate([flat, jnp.full((_NFLATP - n,), -2.0, jnp.float32)])

    bu = lax.bitcast_convert_type(flat_p, jnp.uint32)
    ku = jnp.where(bu >= jnp.uint32(0x80000000), ~bu, bu | jnp.uint32(0x80000000))
    keys_p = lax.bitcast_convert_type(ku ^ jnp.uint32(0x80000000), jnp.int32)

    hist = _sc_hist(keys_p)
    tot = jnp.sum(hist, axis=0)
    rc = jnp.cumsum(tot[::-1])[::-1]
    above = rc - tot
    condv = (above < _PRE_NMS) & (rc >= _PRE_NMS)
    b1 = jnp.argmax(condv).astype(jnp.int32)
    m_cnt = rc[b1]

    bkt_iota = lax.iota(jnp.int32, _NBUCKETS)
    count_t = jnp.sum(jnp.where(bkt_iota[None, :] >= b1, hist, 0), axis=1)
    bases = (jnp.cumsum(count_t) - count_t).astype(jnp.int32)

    kedge = lax.shift_left(b1 - 32768, 16)
    branch_a = (m_cnt <= _CAP) & (b1 >= 32768)

    def f_fast(_):
        kedge16 = jnp.full((16,), kedge, jnp.int32)
        bases_rep = jnp.repeat(bases[:, None], 16, axis=1)
        out_idx = _sc_compact(keys_p, kedge16, bases_rep)[:_CAP]
        slot = lax.iota(jnp.int32, _CAP)
        safe_idx = jnp.clip(out_idx, 0, _NFLATP - 1)
        vals = jnp.where(slot < m_cnt, flat_p[safe_idx], -2.0)
        ts, pos = lax.top_k(vals, _PRE_NMS)
        return ts, out_idx[pos]

    def f_slow(_):
        ts, ti = lax.top_k(flat, _PRE_NMS)
        return ts, ti

    return lax.cond(branch_a, f_fast, f_slow, operand=None)


def _nms_keep_body(rx1, ry1, rx2, ry2, cx1, cy1, cx2, cy2, valid, keep_out):
    n = _PRE_NMS
    m = _BLK
    nb = n // m
    col = lax.broadcasted_iota(jnp.int32, (1, n), 1)
    li = lax.broadcasted_iota(jnp.int32, (m, m), 0)
    lj = lax.broadcasted_iota(jnp.int32, (m, m), 1)
    tri = (li < lj).astype(jnp.float32)

    x1c = cx1[...]
    y1c = cy1[...]
    x2c = cx2[...]
    y2c = cy2[...]
    area_c = jnp.maximum(x2c - x1c, 0.0) * jnp.maximum(y2c - y1c, 0.0)

    keep = valid[...]

    for b in range(nb):
        r0 = b * m
        x1r = rx1[pl.ds(r0, m), :]
        y1r = ry1[pl.ds(r0, m), :]
        x2r = rx2[pl.ds(r0, m), :]
        y2r = ry2[pl.ds(r0, m), :]
        area_r = jnp.maximum(x2r - x1r, 0.0) * jnp.maximum(y2r - y1r, 0.0)
        ltx = jnp.maximum(x1r, x1c)
        lty = jnp.maximum(y1r, y1c)
        rbx = jnp.minimum(x2r, x2c)
        rby = jnp.minimum(y2r, y2c)
        w = jnp.maximum(rbx - ltx, 0.0)
        h = jnp.maximum(rby - lty, 0.0)
        inter = w * h
        union = area_r + area_c - inter
        iou = inter / jnp.maximum(union, 1e-9)
        sup_f = (iou > _IOU_T).astype(jnp.float32)

        sbb = sup_f[:, r0:r0 + m] * tri
        kb0 = keep[:, r0:r0 + m]

        def w_cond(c):
            return c[1]

        def w_body(c, kb0=kb0, sbb=sbb):
            kb, _ = c
            s = lax.dot_general(kb, sbb, (((1,), (0,)), ((), ())),
                                preferred_element_type=jnp.float32)
            kb_new = jnp.where(s > 0.0, 0.0, kb0)
            return kb_new, jnp.any(kb_new != kb)

        kb, _ = lax.while_loop(w_cond, w_body, (kb0, True))

        sup_later = lax.dot_general(kb, sup_f, (((1,), (0,)), ((), ())),
                                    preferred_element_type=jnp.float32)
        pieces = []
        if r0 > 0:
            pieces.append(keep[:, :r0])
        pieces.append(kb)
        if r0 + m < n:
            pieces.append(keep[:, r0 + m:])
        keep = jnp.concatenate(pieces, axis=1) if len(pieces) > 1 else kb
        keep = jnp.where((col >= r0 + m) & (sup_later > 0.0), 0.0, keep)

    keep_out[...] = keep


def _nms_one(bx, sc):
    n_cls = sc.shape[-1]
    flat = sc.reshape(-1)
    flat = jnp.where(flat >= _SCORE_T, flat, -1.0)
    top_s, top_i = _select_topk(flat)
    box_idx = top_i // n_cls
    labels = top_i % n_cls
    cand = bx[box_idx]
    max_c = jnp.max(bx) + 1.0
    off = labels.astype(bx.dtype)[:, None] * max_c
    shifted = cand + off
    valid_f = (top_s > 0.0).astype(jnp.float32)[None, :]

    rows = [shifted[:, i:i + 1] for i in range(4)]
    cols = [shifted[:, i][None, :] for i in range(4)]

    keep_f = pl.pallas_call(
        _nms_keep_body,
        out_shape=jax.ShapeDtypeStruct((1, _PRE_NMS), jnp.float32),
    )(*rows, *cols, valid_f)

    keep = keep_f[0] > 0.5
    kept_scores = jnp.where(keep, top_s, -1.0)
    fs, fi = lax.top_k(kept_scores, _MAX_DET)
    sel_ok = fs > 0.0
    out_boxes = jnp.where(sel_ok[:, None], cand[fi], 0.0)
    out_scores = jnp.where(sel_ok, fs, 0.0)
    out_labels = jnp.where(sel_ok, labels[fi], 0).astype(jnp.int32)
    n_valid = jnp.sum(sel_ok).astype(jnp.int32)
    return out_boxes, out_scores, out_labels, n_valid


def kernel(boxes, scores):
    bdim = boxes.shape[0]
    outs = [_nms_one(boxes[i], scores[i]) for i in range(bdim)]
    return tuple(jnp.stack([o[k] for o in outs]) for k in range(4))


# speedup vs baseline: 427.9805x; 427.9805x over previous
import functools

import jax
import jax.numpy as jnp
from jax import lax
from jax.experimental import pallas as pl
from jax.experimental.pallas import tpu as pltpu
from jax.experimental.pallas import tpu_sc as plsc

_SCORE_T = 0.001
_IOU_T = 0.7
_MAX_DET = 300
_PRE_NMS = 4096
_BLK = 256

_NTILES = 32
_CHUNK = 50176
_NFLATP = _NTILES * _CHUNK
_NBUCKETS = 65536
_CAP = 16384
_ROWS = _CHUNK // 128
_SLAB_ROWS = _ROWS // 2
_SLAB = _SLAB_ROWS * 128


def _sc_hist_body(keys_hbm, hist_hbm, chunk_v, hist_v, sem):
    cid = lax.axis_index("c")
    sid = lax.axis_index("s")
    wid = sid * 2 + cid
    base = wid * _CHUNK

    zeros16 = jnp.zeros((16,), jnp.int32)

    def zbody(j, carry):
        hist_v[pl.ds(j * 16, 16)] = zeros16
        return carry

    lax.fori_loop(0, _NBUCKETS // 16, zbody, 0)

    pltpu.sync_copy(keys_hbm.at[pl.ds(base, _CHUNK)], chunk_v)

    ones16 = jnp.ones((16,), jnp.int32)

    def body(j, carry):
        k = chunk_v[pl.ds(j * 16, 16)]
        bkt = (k >> 16) + 32768
        plsc.addupdate_scatter(hist_v, [bkt], ones16)
        return carry

    lax.fori_loop(0, _CHUNK // 16, body, 0)
    pltpu.sync_copy(hist_v, hist_hbm.at[wid])


def _sc_hist(keys_p):
    mesh = plsc.VectorSubcoreMesh(core_axis_name="c", subcore_axis_name="s")
    return pl.kernel(
        _sc_hist_body,
        out_type=jax.ShapeDtypeStruct((_NTILES, _NBUCKETS), jnp.int32),
        mesh=mesh,
        scratch_types=[
            pltpu.VMEM((_CHUNK,), jnp.int32),
            pltpu.VMEM((_NBUCKETS,), jnp.int32),
            pltpu.SemaphoreType.DMA,
        ],
        compiler_params=pltpu.CompilerParams(needs_layout_passes=False),
    )(keys_p)


def _sc_compact_body(keys_hbm, kedge_hbm, out_hbm,
                     chunk_v, local_v, tvec_v, sem):
    cid = lax.axis_index("c")
    sid = lax.axis_index("s")
    wid = sid * 2 + cid
    base = wid * _CHUNK

    pltpu.sync_copy(kedge_hbm, tvec_v)
    tvec = tvec_v[...]
    lane = lax.iota(jnp.int32, 16)

    pltpu.sync_copy(keys_hbm.at[pl.ds(base, _CHUNK)], chunk_v)

    def body(j, cnt):
        v = chunk_v[pl.ds(j * 16, 16)]
        msk = v >= tvec
        npop = jnp.sum(jnp.where(msk, 1, 0).astype(jnp.int32))
        idxvec = base + j * 16 + lane
        plsc.store_compressed(local_v.at[pl.ds(cnt, 16)], idxvec, mask=msk)
        return cnt + npop

    lax.fori_loop(0, _CHUNK // 16, body, jnp.int32(0))
    pltpu.sync_copy(local_v.at[pl.ds(0, _CAP)], out_hbm.at[wid])


def _sc_compact(keys_p, kedge16):
    mesh = plsc.VectorSubcoreMesh(core_axis_name="c", subcore_axis_name="s")
    return pl.kernel(
        _sc_compact_body,
        out_type=jax.ShapeDtypeStruct((_NTILES, _CAP), jnp.int32),
        mesh=mesh,
        scratch_types=[
            pltpu.VMEM((_CHUNK,), jnp.int32),
            pltpu.VMEM((_CAP + 16,), jnp.int32),
            pltpu.VMEM((16,), jnp.int32),
            pltpu.SemaphoreType.DMA,
        ],
        compiler_params=pltpu.CompilerParams(needs_layout_passes=False),
    )(keys_p, kedge16)


def _select_topk(flat):
    n = flat.shape[0]
    flat_p = jnp.concatenate([flat, jnp.full((_NFLATP - n,), -2.0, jnp.float32)])

    bu = lax.bitcast_convert_type(flat_p, jnp.uint32)
    ku = jnp.where(bu >= jnp.uint32(0x80000000), ~bu, bu | jnp.uint32(0x80000000))
    keys_p = lax.bitcast_convert_type(ku ^ jnp.uint32(0x80000000), jnp.int32)

    hist = _sc_hist(keys_p)
    tot = jnp.sum(hist, axis=0)
    rc = jnp.cumsum(tot[::-1])[::-1]
    above = rc - tot
    condv = (above < _PRE_NMS) & (rc >= _PRE_NMS)
    b1 = jnp.argmax(condv).astype(jnp.int32)
    m_cnt = rc[b1]

    bkt_iota = lax.iota(jnp.int32, _NBUCKETS)
    count_t = jnp.sum(jnp.where(bkt_iota[None, :] >= b1, hist, 0), axis=1)
    bases = (jnp.cumsum(count_t) - count_t).astype(jnp.int32)

    kedge = lax.shift_left(b1 - 32768, 16)
    branch_a = (m_cnt <= _CAP) & (b1 >= 32768)

    def f_fast(_):
        kedge16 = jnp.full((16,), kedge, jnp.int32)
        out2d = _sc_compact(keys_p, kedge16)
        slot = lax.iota(jnp.int32, _CAP)
        t = jnp.searchsorted(bases, slot, side="right").astype(jnp.int32) - 1
        local = slot - bases[t]
        out_idx = out2d[t, jnp.clip(local, 0, _CAP - 1)]
        safe_idx = jnp.clip(out_idx, 0, _NFLATP - 1)
        vals = jnp.where(slot < m_cnt, flat_p[safe_idx], -2.0)
        ts, pos = lax.top_k(vals, _PRE_NMS)
        return ts, out_idx[pos]

    def f_slow(_):
        ts, ti = lax.top_k(flat, _PRE_NMS)
        return ts, ti

    return lax.cond(branch_a, f_fast, f_slow, operand=None)


def _nms_keep_body(rx1, ry1, rx2, ry2, cx1, cy1, cx2, cy2, valid, keep_out):
    n = _PRE_NMS
    m = _BLK
    nb = n // m
    col = lax.broadcasted_iota(jnp.int32, (1, n), 1)
    li = lax.broadcasted_iota(jnp.int32, (m, m), 0)
    lj = lax.broadcasted_iota(jnp.int32, (m, m), 1)
    tri = (li < lj).astype(jnp.float32)

    x1c = cx1[...]
    y1c = cy1[...]
    x2c = cx2[...]
    y2c = cy2[...]
    area_c = jnp.maximum(x2c - x1c, 0.0) * jnp.maximum(y2c - y1c, 0.0)

    keep = valid[...]

    for b in range(nb):
        r0 = b * m
        x1r = rx1[pl.ds(r0, m), :]
        y1r = ry1[pl.ds(r0, m), :]
        x2r = rx2[pl.ds(r0, m), :]
        y2r = ry2[pl.ds(r0, m), :]
        area_r = jnp.maximum(x2r - x1r, 0.0) * jnp.maximum(y2r - y1r, 0.0)
        ltx = jnp.maximum(x1r, x1c)
        lty = jnp.maximum(y1r, y1c)
        rbx = jnp.minimum(x2r, x2c)
        rby = jnp.minimum(y2r, y2c)
        w = jnp.maximum(rbx - ltx, 0.0)
        h = jnp.maximum(rby - lty, 0.0)
        inter = w * h
        union = area_r + area_c - inter
        iou = inter / jnp.maximum(union, 1e-9)
        sup_f = (iou > _IOU_T).astype(jnp.float32)

        sbb = sup_f[:, r0:r0 + m] * tri
        kb0 = keep[:, r0:r0 + m]

        def w_cond(c):
            return c[1]

        def w_body(c, kb0=kb0, sbb=sbb):
            kb, _ = c
            s = lax.dot_general(kb, sbb, (((1,), (0,)), ((), ())),
                                preferred_element_type=jnp.float32)
            kb_new = jnp.where(s > 0.0, 0.0, kb0)
            return kb_new, jnp.any(kb_new != kb)

        kb, _ = lax.while_loop(w_cond, w_body, (kb0, True))

        sup_later = lax.dot_general(kb, sup_f, (((1,), (0,)), ((), ())),
                                    preferred_element_type=jnp.float32)
        pieces = []
        if r0 > 0:
            pieces.append(keep[:, :r0])
        pieces.append(kb)
        if r0 + m < n:
            pieces.append(keep[:, r0 + m:])
        keep = jnp.concatenate(pieces, axis=1) if len(pieces) > 1 else kb
        keep = jnp.where((col >= r0 + m) & (sup_later > 0.0), 0.0, keep)

    keep_out[...] = keep


def _nms_one(bx, sc):
    n_cls = sc.shape[-1]
    flat = sc.reshape(-1)
    flat = jnp.where(flat >= _SCORE_T, flat, -1.0)
    top_s, top_i = _select_topk(flat)
    box_idx = top_i // n_cls
    labels = top_i % n_cls
    cand = bx[box_idx]
    max_c = jnp.max(bx) + 1.0
    off = labels.astype(bx.dtype)[:, None] * max_c
    shifted = cand + off
    valid_f = (top_s > 0.0).astype(jnp.float32)[None, :]

    rows = [shifted[:, i:i + 1] for i in range(4)]
    cols = [shifted[:, i][None, :] for i in range(4)]

    keep_f = pl.pallas_call(
        _nms_keep_body,
        out_shape=jax.ShapeDtypeStruct((1, _PRE_NMS), jnp.float32),
    )(*rows, *cols, valid_f)

    keep = keep_f[0] > 0.5
    kept_scores = jnp.where(keep, top_s, -1.0)
    fs, fi = lax.top_k(kept_scores, _MAX_DET)
    sel_ok = fs > 0.0
    out_boxes = jnp.where(sel_ok[:, None], cand[fi], 0.0)
    out_scores = jnp.where(sel_ok, fs, 0.0)
    out_labels = jnp.where(sel_ok, labels[fi], 0).astype(jnp.int32)
    n_valid = jnp.sum(sel_ok).astype(jnp.int32)
    return out_boxes, out_scores, out_labels, n_valid


def kernel(boxes, scores):
    bdim = boxes.shape[0]
    outs = [_nms_one(boxes[i], scores[i]) for i in range(bdim)]
    return tuple(jnp.stack([o[k] for o in outs]) for k in range(4))
